# baseline (speedup 1.0000x reference)
import jax
import jax.numpy as jnp
from jax import lax
from jax.experimental import pallas as pl

LOOK_BACK = 20
TRADE_K = 64
_INT_MIN = -(2**31)


def _reduce_body(x_ref, cum_ref):
    x = x_ref[0]
    xt = jnp.transpose(x)
    acc = xt[48:49, :]
    for i in range(1, LOOK_BACK):
        acc = acc + xt[48 + 4 * i:49 + 4 * i, :]
    cum_ref[0, 0, :] = acc[0, :]


def _row_count(mask):
    return jnp.sum(mask.astype(jnp.int32), axis=1, keepdims=True)


def _topk_mask(keys, iota, k):
    rows = keys.shape[0]
    t_ub = jnp.zeros((rows, 1), jnp.int32)
    for b in range(31, -1, -1):
        cand_ub = (t_ub | jnp.int32(1 << b)) if b < 31 else (t_ub | _INT_MIN)
        cand_s = cand_ub ^ _INT_MIN
        cnt = _row_count(keys >= cand_s)
        t_ub = jnp.where(cnt >= k, cand_ub, t_ub)
    t_s = t_ub ^ _INT_MIN
    gt = keys > t_s
    need = k - _row_count(gt)
    eq = keys == t_s
    lo = jnp.zeros((rows, 1), jnp.int32)
    for b in range(12, -1, -1):
        c2 = lo + jnp.int32(1 << b)
        pre = _row_count(eq & (iota < c2))
        lo = jnp.where(pre < need, c2, lo)
    return gt | (eq & (iota <= lo))


def _select_body(cum_ref, out_ref):
    x = cum_ref[...]
    x = jnp.where(x == 0.0, 0.0, x)
    i = lax.bitcast_convert_type(x, jnp.int32)
    ks = jnp.where(i >= 0, i, i ^ jnp.int32(0x7FFFFFFF))
    iota = lax.broadcasted_iota(jnp.int32, x.shape, 1)
    w_mask = _topk_mask(ks, iota, TRADE_K)
    l_mask = _topk_mask(~ks, iota, TRADE_K)
    w = jnp.float32(1.0 / (2 * TRADE_K))
    out_ref[...] = jnp.where(w_mask, -w, jnp.where(l_mask, w, 0.0))


def kernel(signal_features):
    bsz, n_assets, n_t, n_f = signal_features.shape
    sf2 = signal_features.reshape(bsz, n_assets, n_t * n_f)
    cum = pl.pallas_call(
        _reduce_body,
        grid=(bsz,),
        in_specs=[pl.BlockSpec((1, n_assets, 128), lambda b: (b, 0, 1))],
        out_specs=pl.BlockSpec((1, 1, n_assets), lambda b: (b, 0, 0)),
        out_shape=jax.ShapeDtypeStruct((bsz, 1, n_assets), jnp.float32),
    )(sf2)
    cum = cum.reshape(bsz, n_assets)
    actions = pl.pallas_call(
        _select_body,
        out_shape=jax.ShapeDtypeStruct((bsz, n_assets), jnp.float32),
    )(cum)
    return (actions, jnp.zeros_like(actions))


# speedup vs baseline: 6.6691x; 6.6691x over previous
import jax
import jax.numpy as jnp
from jax import lax
from jax.experimental import pallas as pl

LOOK_BACK = 20
TRADE_K = 64
_INT_MIN = -(2**31)


def _reduce_body(x_ref, out_ref):
    tj = pl.program_id(1)
    x = x_ref[...]
    bbs = x.shape[0]
    x5 = x.reshape(bbs, 4, 32, 4, 128)
    sel = x5[:, :, :, 0, :]

    @pl.when(tj == 0)
    def _():
        out_ref[...] = jnp.zeros_like(out_ref)

    acc = out_ref[...]
    for k in range(4):
        acc = acc + sel[:, k]
    out_ref[...] = acc


def _row_count(mask):
    return jnp.sum(mask.astype(jnp.int32), axis=1, keepdims=True)


def _topk_mask(keys, iota, k):
    rows = keys.shape[0]
    t_ub = jnp.zeros((rows, 1), jnp.int32)
    for b in range(31, -1, -1):
        cand_ub = (t_ub | jnp.int32(1 << b)) if b < 31 else (t_ub | _INT_MIN)
        cand_s = cand_ub ^ _INT_MIN
        cnt = _row_count(keys >= cand_s)
        t_ub = jnp.where(cnt >= k, cand_ub, t_ub)
    t_s = t_ub ^ _INT_MIN
    gt = keys > t_s
    need = k - _row_count(gt)
    eq = keys == t_s
    lo = jnp.zeros((rows, 1), jnp.int32)
    for b in range(12, -1, -1):
        c2 = lo + jnp.int32(1 << b)
        pre = _row_count(eq & (iota < c2))
        lo = jnp.where(pre < need, c2, lo)
    return gt | (eq & (iota <= lo))


def _select_body(cum_ref, out_ref):
    x3 = cum_ref[...]
    x = x3.reshape(x3.shape[0], x3.shape[1] * x3.shape[2])
    x = jnp.where(x == 0.0, 0.0, x)
    i = lax.bitcast_convert_type(x, jnp.int32)
    ks = jnp.where(i >= 0, i, i ^ jnp.int32(0x7FFFFFFF))
    iota = lax.broadcasted_iota(jnp.int32, x.shape, 1)
    w_mask = _topk_mask(ks, iota, TRADE_K)
    l_mask = _topk_mask(~ks, iota, TRADE_K)
    w = jnp.float32(1.0 / (2 * TRADE_K))
    out_ref[...] = jnp.where(w_mask, -w, jnp.where(l_mask, w, 0.0))


def kernel(signal_features):
    bsz, n_assets, n_t, n_f = signal_features.shape
    ng = n_assets // 128
    view = (signal_features
            .transpose(0, 2, 3, 1)
            .reshape(bsz, n_t, n_f, ng, 128)
            .transpose(0, 1, 3, 2, 4)
            .reshape(bsz, n_t, n_f * ng, 128))
    bb = 4
    t0 = (n_t - LOOK_BACK) // 4
    cum = pl.pallas_call(
        _reduce_body,
        grid=(bsz // bb, LOOK_BACK // 4),
        in_specs=[pl.BlockSpec((bb, 4, n_f * ng, 128),
                               lambda b, t: (b, t0 + t, 0, 0))],
        out_specs=pl.BlockSpec((bb, ng, 128), lambda b, t: (b, 0, 0)),
        out_shape=jax.ShapeDtypeStruct((bsz, ng, 128), jnp.float32),
    )(view)
    actions = pl.pallas_call(
        _select_body,
        out_shape=jax.ShapeDtypeStruct((bsz, n_assets), jnp.float32),
    )(cum)
    return (actions, jnp.zeros_like(actions))


# speedup vs baseline: 9.9037x; 1.4850x over previous
import functools

import jax
import jax.numpy as jnp
from jax import lax
from jax.experimental import pallas as pl
from jax.experimental.pallas import tpu as pltpu
from jax.experimental.pallas import tpu_sc as plsc

LOOK_BACK = 20
TRADE_K = 64
_INT_MIN = -(2**31)


def _reduce_body(x_ref, out_ref):
    tj = pl.program_id(1)
    x = x_ref[...]
    bbs = x.shape[0]
    x5 = x.reshape(bbs, 4, 32, 4, 128)
    sel = x5[:, :, :, 0, :]

    @pl.when(tj == 0)
    def _():
        out_ref[...] = jnp.zeros_like(out_ref)

    acc = out_ref[...]
    for k in range(4):
        acc = acc + sel[:, k]
    out_ref[...] = acc


def _row_count(mask):
    return jnp.sum(mask.astype(jnp.int32), axis=1, keepdims=True)


def _topk_mask(keys, iota, k):
    rows = keys.shape[0]
    t_ub = jnp.zeros((rows, 1), jnp.int32)
    for b in range(31, -1, -1):
        cand_ub = (t_ub | jnp.int32(1 << b)) if b < 31 else (t_ub | _INT_MIN)
        cand_s = cand_ub ^ _INT_MIN
        cnt = _row_count(keys >= cand_s)
        t_ub = jnp.where(cnt >= k, cand_ub, t_ub)
    t_s = t_ub ^ _INT_MIN
    gt = keys > t_s
    need = k - _row_count(gt)
    eq = keys == t_s
    lo = jnp.zeros((rows, 1), jnp.int32)
    for b in range(12, -1, -1):
        c2 = lo + jnp.int32(1 << b)
        pre = _row_count(eq & (iota < c2))
        lo = jnp.where(pre < need, c2, lo)
    return gt | (eq & (iota <= lo))


def _select_body(cum_ref, out_ref):
    x3 = cum_ref[...]
    x = x3.reshape(x3.shape[0], x3.shape[1] * x3.shape[2])
    x = jnp.where(x == 0.0, 0.0, x)
    i = lax.bitcast_convert_type(x, jnp.int32)
    ks = jnp.where(i >= 0, i, i ^ jnp.int32(0x7FFFFFFF))
    iota = lax.broadcasted_iota(jnp.int32, x.shape, 1)
    w_mask = _topk_mask(ks, iota, TRADE_K)
    l_mask = _topk_mask(~ks, iota, TRADE_K)
    w = jnp.float32(1.0 / (2 * TRADE_K))
    out_ref[...] = jnp.where(w_mask, -w, jnp.where(l_mask, w, 0.0))


def _sc_reduce_body(src_ref, out_ref, idx_v, raw_v, acc_v, sem):
    wid = lax.axis_index("s") * 2 + lax.axis_index("c")
    lane = lax.iota(jnp.int32, 16)
    for bi in range(2):
        b = wid * 2 + bi
        for j in range(5):
            for c in range(8):
                k = j * 128 + c * 16 + lane
                t_rel = lax.shift_right_logical(k, 5)
                g = k & 31
                idx_v[j, pl.ds(c * 16, 16)] = (
                    b * 8192 + (44 + t_rel) * 128 + g * 4)
        copies = [
            pltpu.async_copy(src_ref.at[idx_v.at[j]],
                             raw_v.at[pl.ds(j * 128, 128)], sem)
            for j in range(5)
        ]
        for cp in copies:
            cp.wait()

        def _g_body(g, _):
            for c in range(8):
                acc = raw_v[g, pl.ds(c * 16, 16)]
                for t in range(1, LOOK_BACK):
                    acc = acc + raw_v[t * 32 + g, pl.ds(c * 16, 16)]
                acc_v[g, pl.ds(c * 16, 16)] = acc
            return _

        lax.fori_loop(0, 32, _g_body, 0)
        pltpu.sync_copy(acc_v, out_ref.at[b])


def kernel(signal_features):
    bsz, n_assets, n_t, n_f = signal_features.shape
    ng = n_assets // 128
    view = (signal_features
            .transpose(0, 2, 3, 1)
            .reshape(bsz, n_t, n_f, ng, 128)
            .transpose(0, 1, 3, 2, 4)
            .reshape(bsz, n_t, n_f * ng, 128))
    view2d = view.reshape(bsz * n_t * n_f * ng, 128)
    sc_reduce = functools.partial(
        pl.kernel,
        out_type=jax.ShapeDtypeStruct((bsz, ng, 128), jnp.float32),
        mesh=plsc.VectorSubcoreMesh(core_axis_name="c", subcore_axis_name="s"),
        scratch_types=[
            pltpu.VMEM((5, 128), jnp.int32),
            pltpu.VMEM((LOOK_BACK * ng, 128), jnp.float32),
            pltpu.VMEM((ng, 128), jnp.float32),
            pltpu.SemaphoreType.DMA,
        ],
    )(_sc_reduce_body)
    cum = sc_reduce(view2d)
    actions = pl.pallas_call(
        _select_body,
        out_shape=jax.ShapeDtypeStruct((bsz, n_assets), jnp.float32),
    )(cum)
    return (actions, jnp.zeros_like(actions))


# speedup vs baseline: 10.7535x; 1.0858x over previous
import functools

import jax
import jax.numpy as jnp
from jax import lax
from jax.experimental import pallas as pl
from jax.experimental.pallas import tpu as pltpu
from jax.experimental.pallas import tpu_sc as plsc

LOOK_BACK = 20
TRADE_K = 64
_INT_MIN = -(2**31)


def _row_count(mask):
    return jnp.sum(mask.astype(jnp.int32), axis=1, keepdims=True)


def _topk_mask(keys, iota, k):
    rows = keys.shape[0]
    t_ub = jnp.zeros((rows, 1), jnp.int32)
    for b in range(31, -1, -1):
        cand_ub = (t_ub | jnp.int32(1 << b)) if b < 31 else (t_ub | _INT_MIN)
        cand_s = cand_ub ^ _INT_MIN
        cnt = _row_count(keys >= cand_s)
        t_ub = jnp.where(cnt >= k, cand_ub, t_ub)
    t_s = t_ub ^ _INT_MIN
    gt = keys > t_s
    need = k - _row_count(gt)
    eq = keys == t_s
    lo = jnp.zeros((rows, 1), jnp.int32)
    for b in range(12, -1, -1):
        c2 = lo + jnp.int32(1 << b)
        pre = _row_count(eq & (iota < c2))
        lo = jnp.where(pre < need, c2, lo)
    return gt | (eq & (iota <= lo))


def _select_body(cum_ref, out_ref, zero_ref):
    zero_ref[...] = jnp.zeros_like(zero_ref)
    x3 = cum_ref[...]
    x = x3.reshape(x3.shape[0], x3.shape[1] * x3.shape[2])
    x = jnp.where(x == 0.0, 0.0, x)
    i = lax.bitcast_convert_type(x, jnp.int32)
    ks = jnp.where(i >= 0, i, i ^ jnp.int32(0x7FFFFFFF))
    iota = lax.broadcasted_iota(jnp.int32, x.shape, 1)
    w_mask = _topk_mask(ks, iota, TRADE_K)
    l_mask = _topk_mask(~ks, iota, TRADE_K)
    w = jnp.float32(1.0 / (2 * TRADE_K))
    out_ref[...] = jnp.where(w_mask, -w, jnp.where(l_mask, w, 0.0))


def _sc_reduce_body(src_ref, out_ref, idx_v, raw_v, acc_v, sem0, sem1):
    wid = lax.axis_index("s") * 2 + lax.axis_index("c")
    lane = lax.iota(jnp.int32, 16)
    sems = [sem0, sem1]
    half_t = LOOK_BACK // 2

    def build_idx(u):
        b = wid * 2 + u // 2
        p = u % 2
        for j in range(4):
            for c in range(5):
                k = j * 80 + c * 16 + lane
                t_rel = lax.shift_right_logical(k, 5) + (u % 2) * half_t
                g = k & 31
                idx_v[p, j, pl.ds(c * 16, 16)] = (
                    b * 8192 + (44 + t_rel) * 128 + g * 4)

    def fire(u):
        p = u % 2
        return [
            pltpu.async_copy(src_ref.at[idx_v.at[p, j]],
                             raw_v.at[p].at[pl.ds(j * 80, 80)], sems[p])
            for j in range(4)
        ]

    build_idx(0)
    inflight = fire(0)
    for u in range(4):
        if u + 1 < 4:
            build_idx(u + 1)
            nxt = fire(u + 1)
        else:
            nxt = None
        for cp in inflight:
            cp.wait()
        inflight = nxt
        p = u % 2
        h = u % 2

        def _g_body(g, _):
            for c in range(8):
                if h == 0:
                    acc = raw_v[p, g, pl.ds(c * 16, 16)]
                    t_lo = 1
                else:
                    acc = acc_v[g, pl.ds(c * 16, 16)]
                    t_lo = 0
                for t in range(t_lo, half_t):
                    acc = acc + raw_v[p, t * 32 + g, pl.ds(c * 16, 16)]
                acc_v[g, pl.ds(c * 16, 16)] = acc
            return _

        lax.fori_loop(0, 32, _g_body, 0)
        if h == 1:
            pltpu.sync_copy(acc_v, out_ref.at[wid * 2 + u // 2])


def kernel(signal_features):
    bsz, n_assets, n_t, n_f = signal_features.shape
    ng = n_assets // 128
    view = (signal_features
            .transpose(0, 2, 3, 1)
            .reshape(bsz, n_t, n_f, ng, 128)
            .transpose(0, 1, 3, 2, 4)
            .reshape(bsz, n_t, n_f * ng, 128))
    view2d = view.reshape(bsz * n_t * n_f * ng, 128)
    sc_reduce = functools.partial(
        pl.kernel,
        out_type=jax.ShapeDtypeStruct((bsz, ng, 128), jnp.float32),
        mesh=plsc.VectorSubcoreMesh(core_axis_name="c", subcore_axis_name="s"),
        scratch_types=[
            pltpu.VMEM((2, 4, 80), jnp.int32),
            pltpu.VMEM((2, LOOK_BACK // 2 * ng, 128), jnp.float32),
            pltpu.VMEM((ng, 128), jnp.float32),
            pltpu.SemaphoreType.DMA,
            pltpu.SemaphoreType.DMA,
        ],
    )(_sc_reduce_body)
    cum = sc_reduce(view2d)
    actions, zeros = pl.pallas_call(
        _select_body,
        out_shape=(jax.ShapeDtypeStruct((bsz, n_assets), jnp.float32),
                   jax.ShapeDtypeStruct((bsz, n_assets), jnp.float32)),
    )(cum)
    return (actions, zeros)
